# Initial kernel scaffold; baseline (speedup 1.0000x reference)
import jax
import jax.numpy as jnp
from jax.experimental import pallas as pl


def kernel(x, w_gate, expert_w, expert_b, expert_ln_w, expert_ln_b, shared_w, shared_b, shared_ln_w, shared_ln_b):
    raise NotImplementedError("write your pallas kernel here")


# speedup vs baseline: 11.6732x; 11.6732x over previous
import functools

import jax
import jax.numpy as jnp
from jax.experimental import pallas as pl
from jax.experimental.pallas import tpu as pltpu


def _gate_body(x_ref, wg_ref, gates_ref, idx_ref, acc_ref, *, nch, n_exp, inv_hw):
    ch = pl.program_id(1)
    xb = x_ref[0]
    part = jnp.sum(xb, axis=(1, 2), keepdims=True)[:, :, 0]

    @pl.when(ch == 0)
    def _():
        acc_ref[:, :] = part

    @pl.when(ch > 0)
    def _():
        acc_ref[:, :] += part

    @pl.when(ch == nch - 1)
    def _():
        pooled = acc_ref[:, :] * inv_hw
        prod = pooled * wg_ref[:, :]
        logits = jnp.sum(prod, axis=0, keepdims=True)
        ii = jax.lax.broadcasted_iota(jnp.int32, (1, n_exp), 1)
        big_neg = jnp.float32(-1e30)
        m1 = jnp.max(logits, axis=1, keepdims=True)
        a1 = jnp.min(jnp.where(logits == m1, ii, n_exp), axis=1, keepdims=True)
        l2 = jnp.where(ii == a1, big_neg, logits)
        m2 = jnp.max(l2, axis=1, keepdims=True)
        a2 = jnp.min(jnp.where(l2 == m2, ii, n_exp), axis=1, keepdims=True)
        t = jnp.exp(m2 - m1)
        g1 = 1.0 / (1.0 + t)
        g2 = 1.0 - g1
        gates_row = jnp.where(ii == 0, g1,
                              jnp.where(ii == 1, g2,
                                        jnp.where(ii == 2, 1.0, 0.0)))
        idx_row = jnp.where(ii == 0, a1, jnp.where(ii == 1, a2, n_exp))
        gates_ref[0] = gates_row
        idx_ref[0] = idx_row.astype(jnp.int32)


def _moe_body(idx_ref, p_ref, w_ref, effb_ref, lnw_ref, lnb_ref, out_ref,
              acc_ref, *, kt_num):
    kt = pl.program_id(1)
    s = pl.program_id(2)
    part = jnp.dot(p_ref[0], w_ref[0], preferred_element_type=jnp.float32)

    @pl.when(kt == 0)
    def _():
        acc_ref[s] = part

    @pl.when(kt > 0)
    def _():
        acc_ref[s] += part

    @pl.when(kt == kt_num - 1)
    def _():
        y = acc_ref[s] + effb_ref[0, 0]
        u = jnp.mean(y, axis=1, keepdims=True)
        yc = y - u
        v = jnp.mean(yc * yc, axis=1, keepdims=True)
        yn = yc * jax.lax.rsqrt(v + 1e-6)
        yo = lnw_ref[0, 0] * yn + lnb_ref[0, 0]

        @pl.when(s == 0)
        def _():
            out_ref[0] = yo

        @pl.when(s > 0)
        def _():
            out_ref[0] += yo


def kernel(x, w_gate, expert_w, expert_b, expert_ln_w, expert_ln_b,
           shared_w, shared_b, shared_ln_w, shared_ln_b):
    B, C1, H, W = x.shape
    E = expert_w.shape[0]
    C2 = expert_w.shape[1]
    P = expert_w.shape[3]
    nh, nw = H // P, W // P
    NP = nh * nw
    KD = C1 * P * P
    NS = 3

    NCH = 4
    CH = H // NCH
    gates_o, idx_o = pl.pallas_call(
        functools.partial(_gate_body, nch=NCH, n_exp=E, inv_hw=1.0 / (H * W)),
        grid=(B, NCH),
        in_specs=[
            pl.BlockSpec((1, C1, CH, W), lambda b, ch: (b, 0, ch, 0)),
            pl.BlockSpec((C1, E), lambda b, ch: (0, 0)),
        ],
        out_specs=[
            pl.BlockSpec((1, 1, E), lambda b, ch: (b, 0, 0)),
            pl.BlockSpec((1, 1, E), lambda b, ch: (b, 0, 0)),
        ],
        out_shape=[
            jax.ShapeDtypeStruct((B, 1, E), jnp.float32),
            jax.ShapeDtypeStruct((B, 1, E), jnp.int32),
        ],
        scratch_shapes=[pltpu.VMEM((C1, 1), jnp.float32)],
        compiler_params=pltpu.CompilerParams(
            dimension_semantics=("parallel", "arbitrary")),
    )(x, w_gate)

    gates3 = gates_o[:, 0, :NS]
    idx3 = idx_o[:, 0, :NS]

    patches = x.reshape(B, C1, nh, P, nw, P).transpose(0, 2, 4, 1, 3, 5)
    patches = patches.reshape(B, NP, KD)
    w_all = jnp.concatenate(
        [expert_w.reshape(E, C2, KD), shared_w.reshape(1, C2, KD)], axis=0)
    w_all = w_all.transpose(0, 2, 1)
    b_all = jnp.concatenate([expert_b, shared_b[None]], axis=0)
    lnw_all = jnp.concatenate([expert_ln_w, shared_ln_w[None]], axis=0)
    lnb_all = jnp.concatenate([expert_ln_b, shared_ln_b[None]], axis=0)
    eff_b = b_all[idx3].reshape(B, NS, 1, C2)
    eff_lnw = (gates3[..., None] * lnw_all[idx3]).reshape(B, NS, 1, C2)
    eff_lnb = (gates3[..., None] * lnb_all[idx3]).reshape(B, NS, 1, C2)

    BK = 4096
    KT = KD // BK
    grid_spec = pltpu.PrefetchScalarGridSpec(
        num_scalar_prefetch=1,
        grid=(B, KT, NS),
        in_specs=[
            pl.BlockSpec((1, NP, BK), lambda b, kt, s, idx: (b, 0, kt)),
            pl.BlockSpec((1, BK, C2), lambda b, kt, s, idx: (idx[b, s], kt, 0)),
            pl.BlockSpec((1, 1, 1, C2), lambda b, kt, s, idx: (b, s, 0, 0)),
            pl.BlockSpec((1, 1, 1, C2), lambda b, kt, s, idx: (b, s, 0, 0)),
            pl.BlockSpec((1, 1, 1, C2), lambda b, kt, s, idx: (b, s, 0, 0)),
        ],
        out_specs=pl.BlockSpec((1, NP, C2), lambda b, kt, s, idx: (b, 0, 0)),
        scratch_shapes=[pltpu.VMEM((NS, NP, C2), jnp.float32)],
    )
    out = pl.pallas_call(
        functools.partial(_moe_body, kt_num=KT),
        grid_spec=grid_spec,
        out_shape=jax.ShapeDtypeStruct((B, NP, C2), jnp.float32),
        compiler_params=pltpu.CompilerParams(
            dimension_semantics=("parallel", "arbitrary", "arbitrary")),
    )(idx3, patches, w_all, eff_b, eff_lnw, eff_lnb)

    return out.reshape(B, nh, nw, C2).transpose(0, 3, 1, 2)


# speedup vs baseline: 11.7155x; 1.0036x over previous
import functools

import jax
import jax.numpy as jnp
from jax.experimental import pallas as pl
from jax.experimental.pallas import tpu as pltpu


def _gate_body(x_ref, wg_ref, gates_ref, idx_ref, acc_ref, *, nch, n_exp, inv_hw):
    ch = pl.program_id(1)
    xb = x_ref[0]
    part = jnp.sum(xb, axis=(1, 2), keepdims=True)[:, :, 0]

    @pl.when(ch == 0)
    def _():
        acc_ref[:, :] = part

    @pl.when(ch > 0)
    def _():
        acc_ref[:, :] += part

    @pl.when(ch == nch - 1)
    def _():
        pooled = acc_ref[:, :] * inv_hw
        prod = pooled * wg_ref[:, :]
        logits = jnp.sum(prod, axis=0, keepdims=True)
        ii = jax.lax.broadcasted_iota(jnp.int32, (1, n_exp), 1)
        big_neg = jnp.float32(-1e30)
        m1 = jnp.max(logits, axis=1, keepdims=True)
        a1 = jnp.min(jnp.where(logits == m1, ii, n_exp), axis=1, keepdims=True)
        l2 = jnp.where(ii == a1, big_neg, logits)
        m2 = jnp.max(l2, axis=1, keepdims=True)
        a2 = jnp.min(jnp.where(l2 == m2, ii, n_exp), axis=1, keepdims=True)
        t = jnp.exp(m2 - m1)
        g1 = 1.0 / (1.0 + t)
        g2 = 1.0 - g1
        gates_row = jnp.where(ii == 0, g1,
                              jnp.where(ii == 1, g2,
                                        jnp.where(ii == 2, 1.0, 0.0)))
        idx_row = jnp.where(ii == 0, a1, jnp.where(ii == 1, a2, n_exp))
        gates_ref[0] = gates_row
        idx_ref[0] = idx_row.astype(jnp.int32)


def _moe_body(idx_ref, p_ref, w_ref, wsh_ref, effb_ref, lnw_ref, lnb_ref,
              out_ref, acc_ref, *, kt_num):
    kt = pl.program_id(1)
    s = pl.program_id(2)
    wsel = jnp.where(s == 2, wsh_ref[...], w_ref[0])
    part = jax.lax.dot_general(
        p_ref[0], wsel, (((1,), (1,)), ((), ())),
        preferred_element_type=jnp.float32)

    @pl.when(kt == 0)
    def _():
        acc_ref[s] = part

    @pl.when(kt > 0)
    def _():
        acc_ref[s] += part

    @pl.when(kt == kt_num - 1)
    def _():
        y = acc_ref[s] + effb_ref[0, 0]
        u = jnp.mean(y, axis=1, keepdims=True)
        yc = y - u
        v = jnp.mean(yc * yc, axis=1, keepdims=True)
        yn = yc * jax.lax.rsqrt(v + 1e-6)
        yo = lnw_ref[0, 0] * yn + lnb_ref[0, 0]

        @pl.when(s == 0)
        def _():
            out_ref[0] = yo

        @pl.when(s > 0)
        def _():
            out_ref[0] += yo


def kernel(x, w_gate, expert_w, expert_b, expert_ln_w, expert_ln_b,
           shared_w, shared_b, shared_ln_w, shared_ln_b):
    B, C1, H, W = x.shape
    E = expert_w.shape[0]
    C2 = expert_w.shape[1]
    P = expert_w.shape[3]
    nh, nw = H // P, W // P
    NP = nh * nw
    KD = C1 * P * P
    NS = 3

    NCH = 4
    CH = H // NCH
    gates_o, idx_o = pl.pallas_call(
        functools.partial(_gate_body, nch=NCH, n_exp=E, inv_hw=1.0 / (H * W)),
        grid=(B, NCH),
        in_specs=[
            pl.BlockSpec((1, C1, CH, W), lambda b, ch: (b, 0, ch, 0)),
            pl.BlockSpec((C1, E), lambda b, ch: (0, 0)),
        ],
        out_specs=[
            pl.BlockSpec((1, 1, E), lambda b, ch: (b, 0, 0)),
            pl.BlockSpec((1, 1, E), lambda b, ch: (b, 0, 0)),
        ],
        out_shape=[
            jax.ShapeDtypeStruct((B, 1, E), jnp.float32),
            jax.ShapeDtypeStruct((B, 1, E), jnp.int32),
        ],
        scratch_shapes=[pltpu.VMEM((C1, 1), jnp.float32)],
        compiler_params=pltpu.CompilerParams(
            dimension_semantics=("parallel", "arbitrary")),
    )(x, w_gate)

    gates3 = gates_o[:, 0, :NS]
    idx3 = idx_o[:, 0, :NS]

    patches = x.reshape(B, C1, nh, P, nw, P).transpose(0, 2, 4, 1, 3, 5)
    patches = patches.reshape(B, NP, KD)
    w_e = expert_w.reshape(E, C2, KD)
    w_sh = shared_w.reshape(C2, KD)
    b_all = jnp.concatenate([expert_b, shared_b[None]], axis=0)
    lnw_all = jnp.concatenate([expert_ln_w, shared_ln_w[None]], axis=0)
    lnb_all = jnp.concatenate([expert_ln_b, shared_ln_b[None]], axis=0)
    eff_b = b_all[idx3].reshape(B, NS, 1, C2)
    eff_lnw = (gates3[..., None] * lnw_all[idx3]).reshape(B, NS, 1, C2)
    eff_lnb = (gates3[..., None] * lnb_all[idx3]).reshape(B, NS, 1, C2)
    idx_w = jnp.concatenate([idx3[:, :2], idx3[:, 1:2]], axis=1)

    BK = 4096
    KT = KD // BK
    grid_spec = pltpu.PrefetchScalarGridSpec(
        num_scalar_prefetch=1,
        grid=(B, KT, NS),
        in_specs=[
            pl.BlockSpec((1, NP, BK), lambda b, kt, s, idx: (b, 0, kt)),
            pl.BlockSpec((1, C2, BK), lambda b, kt, s, idx: (idx[b, s], 0, kt)),
            pl.BlockSpec((C2, BK), lambda b, kt, s, idx: (0, kt)),
            pl.BlockSpec((1, 1, 1, C2), lambda b, kt, s, idx: (b, s, 0, 0)),
            pl.BlockSpec((1, 1, 1, C2), lambda b, kt, s, idx: (b, s, 0, 0)),
            pl.BlockSpec((1, 1, 1, C2), lambda b, kt, s, idx: (b, s, 0, 0)),
        ],
        out_specs=pl.BlockSpec((1, NP, C2), lambda b, kt, s, idx: (b, 0, 0)),
        scratch_shapes=[pltpu.VMEM((NS, NP, C2), jnp.float32)],
    )
    out = pl.pallas_call(
        functools.partial(_moe_body, kt_num=KT),
        grid_spec=grid_spec,
        out_shape=jax.ShapeDtypeStruct((B, NP, C2), jnp.float32),
        compiler_params=pltpu.CompilerParams(
            dimension_semantics=("parallel", "arbitrary", "arbitrary")),
    )(idx_w, patches, w_e, w_sh, eff_b, eff_lnw, eff_lnb)

    return out.reshape(B, nh, nw, C2).transpose(0, 3, 1, 2)


# speedup vs baseline: 25.9739x; 2.2171x over previous
import functools

import jax
import jax.numpy as jnp
from jax.experimental import pallas as pl
from jax.experimental.pallas import tpu as pltpu


def _gate_body(x_ref, wg_ref, patches_ref, gates_ref, idx_ref, acc_ref, *,
               nch, n_exp, inv_hw, npw, pp):
    ch = pl.program_id(1)
    xb = x_ref[0]
    c1 = xb.shape[0]
    t = xb.reshape(c1 * pp, npw, pp).transpose(1, 0, 2)
    patches_ref[0, 0] = t.reshape(npw, c1 * pp * pp)
    part = jnp.sum(xb, axis=(1, 2), keepdims=True)[:, :, 0]

    @pl.when(ch == 0)
    def _():
        acc_ref[:, :] = part

    @pl.when(ch > 0)
    def _():
        acc_ref[:, :] += part

    @pl.when(ch == nch - 1)
    def _():
        pooled = acc_ref[:, :] * inv_hw
        prod = pooled * wg_ref[:, :]
        logits = jnp.sum(prod, axis=0, keepdims=True)
        ii = jax.lax.broadcasted_iota(jnp.int32, (1, n_exp), 1)
        big_neg = jnp.float32(-1e30)
        m1 = jnp.max(logits, axis=1, keepdims=True)
        a1 = jnp.min(jnp.where(logits == m1, ii, n_exp), axis=1, keepdims=True)
        l2 = jnp.where(ii == a1, big_neg, logits)
        m2 = jnp.max(l2, axis=1, keepdims=True)
        a2 = jnp.min(jnp.where(l2 == m2, ii, n_exp), axis=1, keepdims=True)
        t = jnp.exp(m2 - m1)
        g1 = 1.0 / (1.0 + t)
        g2 = 1.0 - g1
        gates_row = jnp.where(ii == 0, g1,
                              jnp.where(ii == 1, g2,
                                        jnp.where(ii == 2, 1.0, 0.0)))
        idx_row = jnp.where(ii == 0, a1, jnp.where(ii == 1, a2, n_exp))
        gates_ref[0] = gates_row
        idx_ref[0] = idx_row.astype(jnp.int32)


def _moe_body(idx_ref, p_ref, w_ref, wsh_ref, effb_ref, lnw_ref, lnb_ref,
              out_ref, acc_ref, *, kt_num):
    kt = pl.program_id(1)
    s = pl.program_id(2)
    wsel = jnp.where(s == 2, wsh_ref[...], w_ref[0])
    part = jax.lax.dot_general(
        p_ref[0], wsel, (((1,), (1,)), ((), ())),
        preferred_element_type=jnp.float32)

    @pl.when(kt == 0)
    def _():
        acc_ref[s] = part

    @pl.when(kt > 0)
    def _():
        acc_ref[s] += part

    @pl.when(kt == kt_num - 1)
    def _():
        y = acc_ref[s] + effb_ref[0, 0]
        u = jnp.mean(y, axis=1, keepdims=True)
        yc = y - u
        v = jnp.mean(yc * yc, axis=1, keepdims=True)
        yn = yc * jax.lax.rsqrt(v + 1e-6)
        yo = lnw_ref[0, 0] * yn + lnb_ref[0, 0]

        @pl.when(s == 0)
        def _():
            out_ref[0] = yo

        @pl.when(s > 0)
        def _():
            out_ref[0] += yo


def kernel(x, w_gate, expert_w, expert_b, expert_ln_w, expert_ln_b,
           shared_w, shared_b, shared_ln_w, shared_ln_b):
    B, C1, H, W = x.shape
    E = expert_w.shape[0]
    C2 = expert_w.shape[1]
    P = expert_w.shape[3]
    nh, nw = H // P, W // P
    NP = nh * nw
    KD = C1 * P * P
    NS = 3

    NCH = nh
    patches_o, gates_o, idx_o = pl.pallas_call(
        functools.partial(_gate_body, nch=NCH, n_exp=E, inv_hw=1.0 / (H * W),
                          npw=nw, pp=P),
        grid=(B, NCH),
        in_specs=[
            pl.BlockSpec((1, C1, P, W), lambda b, ch: (b, 0, ch, 0)),
            pl.BlockSpec((C1, E), lambda b, ch: (0, 0)),
        ],
        out_specs=[
            pl.BlockSpec((1, 1, nw, KD), lambda b, ch: (b, ch, 0, 0)),
            pl.BlockSpec((1, 1, E), lambda b, ch: (b, 0, 0)),
            pl.BlockSpec((1, 1, E), lambda b, ch: (b, 0, 0)),
        ],
        out_shape=[
            jax.ShapeDtypeStruct((B, nh, nw, KD), jnp.float32),
            jax.ShapeDtypeStruct((B, 1, E), jnp.float32),
            jax.ShapeDtypeStruct((B, 1, E), jnp.int32),
        ],
        scratch_shapes=[pltpu.VMEM((C1, 1), jnp.float32)],
        compiler_params=pltpu.CompilerParams(
            dimension_semantics=("parallel", "arbitrary")),
    )(x, w_gate)

    gates3 = gates_o[:, 0, :NS]
    idx3 = idx_o[:, 0, :NS]

    patches = patches_o.reshape(B, NP, KD)
    w_e = expert_w.reshape(E, C2, KD)
    w_sh = shared_w.reshape(C2, KD)
    b_all = jnp.concatenate([expert_b, shared_b[None]], axis=0)
    lnw_all = jnp.concatenate([expert_ln_w, shared_ln_w[None]], axis=0)
    lnb_all = jnp.concatenate([expert_ln_b, shared_ln_b[None]], axis=0)
    eff_b = b_all[idx3].reshape(B, NS, 1, C2)
    eff_lnw = (gates3[..., None] * lnw_all[idx3]).reshape(B, NS, 1, C2)
    eff_lnb = (gates3[..., None] * lnb_all[idx3]).reshape(B, NS, 1, C2)
    idx_w = jnp.concatenate([idx3[:, :2], idx3[:, 1:2]], axis=1)

    BK = 4096
    KT = KD // BK
    grid_spec = pltpu.PrefetchScalarGridSpec(
        num_scalar_prefetch=1,
        grid=(B, KT, NS),
        in_specs=[
            pl.BlockSpec((1, NP, BK), lambda b, kt, s, idx: (b, 0, kt)),
            pl.BlockSpec((1, C2, BK), lambda b, kt, s, idx: (idx[b, s], 0, kt)),
            pl.BlockSpec((C2, BK), lambda b, kt, s, idx: (0, kt)),
            pl.BlockSpec((1, 1, 1, C2), lambda b, kt, s, idx: (b, s, 0, 0)),
            pl.BlockSpec((1, 1, 1, C2), lambda b, kt, s, idx: (b, s, 0, 0)),
            pl.BlockSpec((1, 1, 1, C2), lambda b, kt, s, idx: (b, s, 0, 0)),
        ],
        out_specs=pl.BlockSpec((1, NP, C2), lambda b, kt, s, idx: (b, 0, 0)),
        scratch_shapes=[pltpu.VMEM((NS, NP, C2), jnp.float32)],
    )
    out = pl.pallas_call(
        functools.partial(_moe_body, kt_num=KT),
        grid_spec=grid_spec,
        out_shape=jax.ShapeDtypeStruct((B, NP, C2), jnp.float32),
        compiler_params=pltpu.CompilerParams(
            dimension_semantics=("parallel", "arbitrary", "arbitrary")),
    )(idx_w, patches, w_e, w_sh, eff_b, eff_lnw, eff_lnb)

    return out.reshape(B, nh, nw, C2).transpose(0, 3, 1, 2)


# speedup vs baseline: 31.7998x; 1.2243x over previous
import functools

import jax
import jax.numpy as jnp
from jax.experimental import pallas as pl
from jax.experimental.pallas import tpu as pltpu


def _gate_body(x_ref, wg_ref, patches_ref, gates_ref, idx_ref, acc_ref, *,
               nch, n_exp, inv_hw, npw, pp):
    ch = pl.program_id(1)
    xb = x_ref[0]
    c1 = xb.shape[0]
    t = xb.reshape(c1 * pp, npw, pp).transpose(1, 0, 2)
    t = t.reshape(npw, c1 * pp * pp)
    patches_ref[0] = jnp.concatenate(
        [t, jnp.zeros((16 - npw, t.shape[1]), t.dtype)], axis=0)
    part = jnp.sum(xb, axis=(1, 2), keepdims=True)[:, :, 0]

    @pl.when(ch == 0)
    def _():
        acc_ref[:, :] = part

    @pl.when(ch > 0)
    def _():
        acc_ref[:, :] += part

    @pl.when(ch == nch - 1)
    def _():
        pooled = acc_ref[:, :] * inv_hw
        prod = pooled * wg_ref[:, :]
        logits = jnp.sum(prod, axis=0, keepdims=True)
        ii = jax.lax.broadcasted_iota(jnp.int32, (1, n_exp), 1)
        big_neg = jnp.float32(-1e30)
        m1 = jnp.max(logits, axis=1, keepdims=True)
        a1 = jnp.min(jnp.where(logits == m1, ii, n_exp), axis=1, keepdims=True)
        l2 = jnp.where(ii == a1, big_neg, logits)
        m2 = jnp.max(l2, axis=1, keepdims=True)
        a2 = jnp.min(jnp.where(l2 == m2, ii, n_exp), axis=1, keepdims=True)
        t = jnp.exp(m2 - m1)
        g1 = 1.0 / (1.0 + t)
        g2 = 1.0 - g1
        gates_row = jnp.where(ii == 0, g1,
                              jnp.where(ii == 1, g2,
                                        jnp.where(ii == 2, 1.0, 0.0)))
        idx_row = jnp.where(ii == 0, a1, jnp.where(ii == 1, a2, n_exp))
        gates_ref[0] = gates_row
        idx_ref[0] = idx_row.astype(jnp.int32)


def _moe_body(idx_ref, p_ref, w_ref, wsh_ref, effb_ref, lnw_ref, lnb_ref,
              out_ref, acc_ref, *, kt_num):
    kt = pl.program_id(1)
    s = pl.program_id(2)
    wsel = jnp.where(s == 2, wsh_ref[...], w_ref[0])
    part = jax.lax.dot_general(
        p_ref[0], wsel, (((1,), (1,)), ((), ())),
        preferred_element_type=jnp.float32)

    @pl.when(kt == 0)
    def _():
        acc_ref[s] = part

    @pl.when(kt > 0)
    def _():
        acc_ref[s] += part

    @pl.when(kt == kt_num - 1)
    def _():
        y = acc_ref[s] + effb_ref[0, 0]
        u = jnp.mean(y, axis=1, keepdims=True)
        yc = y - u
        v = jnp.mean(yc * yc, axis=1, keepdims=True)
        yn = yc * jax.lax.rsqrt(v + 1e-6)
        yo = lnw_ref[0, 0] * yn + lnb_ref[0, 0]

        @pl.when(s == 0)
        def _():
            out_ref[0] = yo

        @pl.when(s > 0)
        def _():
            out_ref[0] += yo


def kernel(x, w_gate, expert_w, expert_b, expert_ln_w, expert_ln_b,
           shared_w, shared_b, shared_ln_w, shared_ln_b):
    B, C1, H, W = x.shape
    E = expert_w.shape[0]
    C2 = expert_w.shape[1]
    P = expert_w.shape[3]
    nh, nw = H // P, W // P
    NP = nh * nw
    KD = C1 * P * P
    NS = 3

    NCH = nh
    patches_o, gates_o, idx_o = pl.pallas_call(
        functools.partial(_gate_body, nch=NCH, n_exp=E, inv_hw=1.0 / (H * W),
                          npw=nw, pp=P),
        grid=(B, NCH),
        in_specs=[
            pl.BlockSpec((1, C1, P, W), lambda b, ch: (b, 0, ch, 0)),
            pl.BlockSpec((C1, E), lambda b, ch: (0, 0)),
        ],
        out_specs=[
            pl.BlockSpec((1, 16, KD), lambda b, ch: (b, ch, 0)),
            pl.BlockSpec((1, 1, E), lambda b, ch: (b, 0, 0)),
            pl.BlockSpec((1, 1, E), lambda b, ch: (b, 0, 0)),
        ],
        out_shape=[
            jax.ShapeDtypeStruct((B, nh * 16, KD), jnp.float32),
            jax.ShapeDtypeStruct((B, 1, E), jnp.float32),
            jax.ShapeDtypeStruct((B, 1, E), jnp.int32),
        ],
        scratch_shapes=[pltpu.VMEM((C1, 1), jnp.float32)],
        compiler_params=pltpu.CompilerParams(
            dimension_semantics=("parallel", "arbitrary")),
    )(x, w_gate)

    gates3 = gates_o[:, 0, :NS]
    idx3 = idx_o[:, 0, :NS]

    patches = patches_o
    NPP = nh * 16
    w_e = expert_w.reshape(E, C2, KD)
    w_sh = shared_w.reshape(C2, KD)
    b_all = jnp.concatenate([expert_b, shared_b[None]], axis=0)
    lnw_all = jnp.concatenate([expert_ln_w, shared_ln_w[None]], axis=0)
    lnb_all = jnp.concatenate([expert_ln_b, shared_ln_b[None]], axis=0)
    eff_b = b_all[idx3].reshape(B, NS, 1, C2)
    eff_lnw = (gates3[..., None] * lnw_all[idx3]).reshape(B, NS, 1, C2)
    eff_lnb = (gates3[..., None] * lnb_all[idx3]).reshape(B, NS, 1, C2)
    idx_w = jnp.concatenate([idx3[:, :2], idx3[:, 1:2]], axis=1)

    BK = 4096
    KT = KD // BK
    grid_spec = pltpu.PrefetchScalarGridSpec(
        num_scalar_prefetch=1,
        grid=(B, KT, NS),
        in_specs=[
            pl.BlockSpec((1, NPP, BK), lambda b, kt, s, idx: (b, 0, kt)),
            pl.BlockSpec((1, C2, BK), lambda b, kt, s, idx: (idx[b, s], 0, kt)),
            pl.BlockSpec((C2, BK), lambda b, kt, s, idx: (0, kt)),
            pl.BlockSpec((1, 1, 1, C2), lambda b, kt, s, idx: (b, s, 0, 0)),
            pl.BlockSpec((1, 1, 1, C2), lambda b, kt, s, idx: (b, s, 0, 0)),
            pl.BlockSpec((1, 1, 1, C2), lambda b, kt, s, idx: (b, s, 0, 0)),
        ],
        out_specs=pl.BlockSpec((1, NPP, C2), lambda b, kt, s, idx: (b, 0, 0)),
        scratch_shapes=[pltpu.VMEM((NS, NPP, C2), jnp.float32)],
    )
    out = pl.pallas_call(
        functools.partial(_moe_body, kt_num=KT),
        grid_spec=grid_spec,
        out_shape=jax.ShapeDtypeStruct((B, NPP, C2), jnp.float32),
        compiler_params=pltpu.CompilerParams(
            dimension_semantics=("parallel", "arbitrary", "arbitrary")),
    )(idx_w, patches, w_e, w_sh, eff_b, eff_lnw, eff_lnb)

    out = out.reshape(B, nh, 16, C2)[:, :, :nw]
    return out.transpose(0, 3, 1, 2)


# speedup vs baseline: 31.9694x; 1.0053x over previous
import functools

import jax
import jax.numpy as jnp
from jax.experimental import pallas as pl
from jax.experimental.pallas import tpu as pltpu


def _gate_body(x_ref, wg_ref, patches_ref, gates_ref, idx_ref, acc_ref, *,
               nch, n_exp, inv_hw, npw, pp):
    ch = pl.program_id(1)
    xb = x_ref[0]
    c1 = xb.shape[0]
    a = xb.reshape(c1 * pp, npw * pp)
    tt = a.T.reshape(npw, pp, c1 * pp)
    t = tt.transpose(0, 2, 1).reshape(npw, c1 * pp * pp)
    patches_ref[0] = jnp.concatenate(
        [t, jnp.zeros((16 - npw, t.shape[1]), t.dtype)], axis=0)
    part = jnp.sum(xb, axis=(1, 2), keepdims=True)[:, :, 0]

    @pl.when(ch == 0)
    def _():
        acc_ref[:, :] = part

    @pl.when(ch > 0)
    def _():
        acc_ref[:, :] += part

    @pl.when(ch == nch - 1)
    def _():
        pooled = acc_ref[:, :] * inv_hw
        prod = pooled * wg_ref[:, :]
        logits = jnp.sum(prod, axis=0, keepdims=True)
        ii = jax.lax.broadcasted_iota(jnp.int32, (1, n_exp), 1)
        big_neg = jnp.float32(-1e30)
        m1 = jnp.max(logits, axis=1, keepdims=True)
        a1 = jnp.min(jnp.where(logits == m1, ii, n_exp), axis=1, keepdims=True)
        l2 = jnp.where(ii == a1, big_neg, logits)
        m2 = jnp.max(l2, axis=1, keepdims=True)
        a2 = jnp.min(jnp.where(l2 == m2, ii, n_exp), axis=1, keepdims=True)
        t = jnp.exp(m2 - m1)
        g1 = 1.0 / (1.0 + t)
        g2 = 1.0 - g1
        gates_row = jnp.where(ii == 0, g1,
                              jnp.where(ii == 1, g2,
                                        jnp.where(ii == 2, 1.0, 0.0)))
        idx_row = jnp.where(ii == 0, a1, jnp.where(ii == 1, a2, n_exp))
        gates_ref[0] = gates_row
        idx_ref[0] = idx_row.astype(jnp.int32)


def _moe_body(idx_ref, p_ref, w_ref, wsh_ref, effb_ref, lnw_ref, lnb_ref,
              out_ref, acc_ref, *, kt_num):
    kt = pl.program_id(1)
    s = pl.program_id(2)
    wsel = jnp.where(s == 2, wsh_ref[...], w_ref[0])
    part = jax.lax.dot_general(
        p_ref[0], wsel, (((1,), (1,)), ((), ())),
        preferred_element_type=jnp.float32)

    @pl.when(kt == 0)
    def _():
        acc_ref[s] = part

    @pl.when(kt > 0)
    def _():
        acc_ref[s] += part

    @pl.when(kt == kt_num - 1)
    def _():
        y = acc_ref[s] + effb_ref[0, 0]
        u = jnp.mean(y, axis=1, keepdims=True)
        yc = y - u
        v = jnp.mean(yc * yc, axis=1, keepdims=True)
        yn = yc * jax.lax.rsqrt(v + 1e-6)
        yo = lnw_ref[0, 0] * yn + lnb_ref[0, 0]

        @pl.when(s == 0)
        def _():
            out_ref[0] = yo

        @pl.when(s > 0)
        def _():
            out_ref[0] += yo


def kernel(x, w_gate, expert_w, expert_b, expert_ln_w, expert_ln_b,
           shared_w, shared_b, shared_ln_w, shared_ln_b):
    B, C1, H, W = x.shape
    E = expert_w.shape[0]
    C2 = expert_w.shape[1]
    P = expert_w.shape[3]
    nh, nw = H // P, W // P
    NP = nh * nw
    KD = C1 * P * P
    NS = 3

    NCH = nh
    patches_o, gates_o, idx_o = pl.pallas_call(
        functools.partial(_gate_body, nch=NCH, n_exp=E, inv_hw=1.0 / (H * W),
                          npw=nw, pp=P),
        grid=(B, NCH),
        in_specs=[
            pl.BlockSpec((1, C1, P, W), lambda b, ch: (b, 0, ch, 0)),
            pl.BlockSpec((C1, E), lambda b, ch: (0, 0)),
        ],
        out_specs=[
            pl.BlockSpec((1, 16, KD), lambda b, ch: (b, ch, 0)),
            pl.BlockSpec((1, 1, E), lambda b, ch: (b, 0, 0)),
            pl.BlockSpec((1, 1, E), lambda b, ch: (b, 0, 0)),
        ],
        out_shape=[
            jax.ShapeDtypeStruct((B, nh * 16, KD), jnp.float32),
            jax.ShapeDtypeStruct((B, 1, E), jnp.float32),
            jax.ShapeDtypeStruct((B, 1, E), jnp.int32),
        ],
        scratch_shapes=[pltpu.VMEM((C1, 1), jnp.float32)],
        compiler_params=pltpu.CompilerParams(
            dimension_semantics=("parallel", "arbitrary")),
    )(x, w_gate)

    gates3 = gates_o[:, 0, :NS]
    idx3 = idx_o[:, 0, :NS]

    patches = patches_o
    NPP = nh * 16
    w_e = expert_w.reshape(E, C2, KD)
    w_sh = shared_w.reshape(C2, KD)
    b_all = jnp.concatenate([expert_b, shared_b[None]], axis=0)
    lnw_all = jnp.concatenate([expert_ln_w, shared_ln_w[None]], axis=0)
    lnb_all = jnp.concatenate([expert_ln_b, shared_ln_b[None]], axis=0)
    eff_b = b_all[idx3].reshape(B, NS, 1, C2)
    eff_lnw = (gates3[..., None] * lnw_all[idx3]).reshape(B, NS, 1, C2)
    eff_lnb = (gates3[..., None] * lnb_all[idx3]).reshape(B, NS, 1, C2)
    idx_w = jnp.concatenate([idx3[:, :2], idx3[:, 1:2]], axis=1)

    BK = 4096
    KT = KD // BK
    grid_spec = pltpu.PrefetchScalarGridSpec(
        num_scalar_prefetch=1,
        grid=(B, KT, NS),
        in_specs=[
            pl.BlockSpec((1, NPP, BK), lambda b, kt, s, idx: (b, 0, kt)),
            pl.BlockSpec((1, C2, BK), lambda b, kt, s, idx: (idx[b, s], 0, kt)),
            pl.BlockSpec((C2, BK), lambda b, kt, s, idx: (0, kt)),
            pl.BlockSpec((1, 1, 1, C2), lambda b, kt, s, idx: (b, s, 0, 0)),
            pl.BlockSpec((1, 1, 1, C2), lambda b, kt, s, idx: (b, s, 0, 0)),
            pl.BlockSpec((1, 1, 1, C2), lambda b, kt, s, idx: (b, s, 0, 0)),
        ],
        out_specs=pl.BlockSpec((1, NPP, C2), lambda b, kt, s, idx: (b, 0, 0)),
        scratch_shapes=[pltpu.VMEM((NS, NPP, C2), jnp.float32)],
    )
    out = pl.pallas_call(
        functools.partial(_moe_body, kt_num=KT),
        grid_spec=grid_spec,
        out_shape=jax.ShapeDtypeStruct((B, NPP, C2), jnp.float32),
        compiler_params=pltpu.CompilerParams(
            dimension_semantics=("parallel", "arbitrary", "arbitrary")),
    )(idx_w, patches, w_e, w_sh, eff_b, eff_lnw, eff_lnb)

    out = out.reshape(B, nh, 16, C2)[:, :, :nw]
    return out.transpose(0, 3, 1, 2)
